# baseline (speedup 1.0000x reference)
import functools

import jax
import jax.numpy as jnp
from jax import lax
from jax.experimental import pallas as pl
from jax.experimental.pallas import tpu as pltpu
from jax.experimental.pallas import tpu_sc as plsc

N, E, D, H, C, G = 10000, 320000, 128, 32, 2, 64

NC, NS = 2, 16
NW = NC * NS
K = 128
NB = 4
CH = NB * (-(-E // (NW * K * NB)))
E_PAD = NW * CH * K
N_PAD = N + 8


def _segsum_body(y_hbm, src_hbm, dst_hbm, zeros_hbm, out_hbm,
                 src_v, dst_v, rows_v, acc, y_sh, gsems):
    c = lax.axis_index("c")
    s = lax.axis_index("s")
    wid = s * NC + c

    @pl.when(s == 0)
    def _zero():
        pltpu.sync_copy(zeros_hbm, acc)

    @pl.when(s == 1)
    def _stage():
        pltpu.sync_copy(y_hbm, y_sh)

    plsc.subcore_barrier()

    pltpu.sync_copy(src_hbm.at[wid], src_v)
    pltpu.sync_copy(dst_hbm.at[wid], dst_v)

    for b in range(NB):
        pltpu.async_copy(y_sh.at[src_v.at[b]], rows_v.at[b], gsems[b])

    def body(g, carry):
        for b in range(NB):
            j = g * NB + b
            pltpu.make_async_copy(y_sh.at[src_v.at[j]], rows_v.at[b],
                                  gsems[b]).wait()
            pltpu.sync_copy(rows_v.at[b], acc.at[dst_v.at[j]], add=True)

            @pl.when(j + NB < CH)
            def _refill():
                pltpu.async_copy(y_sh.at[src_v.at[j + NB]], rows_v.at[b],
                                 gsems[b])
        return carry

    lax.fori_loop(0, CH // NB, body, 0)

    plsc.subcore_barrier()

    @pl.when(s == 0)
    def _writeout():
        pltpu.sync_copy(acc.at[pl.ds(0, N)], out_hbm.at[c])


_segsum = functools.partial(
    pl.kernel,
    out_type=jax.ShapeDtypeStruct((2, N, H), jnp.float32),
    mesh=plsc.VectorSubcoreMesh(core_axis_name="c", subcore_axis_name="s",
                                num_cores=NC, num_subcores=NS),
    compiler_params=pltpu.CompilerParams(use_tc_tiling_on_sc=False),
    scratch_types=[
        pltpu.VMEM((CH, K), jnp.int32),
        pltpu.VMEM((CH, K), jnp.int32),
        pltpu.VMEM((NB, K, H), jnp.float32),
        pltpu.VMEM_SHARED((N_PAD, H), jnp.float32),
        pltpu.VMEM_SHARED((N, H), jnp.float32),
        [pltpu.SemaphoreType.DMA] * NB,
    ],
)(_segsum_body)


def _proj_body(x_ref, w_ref, o_ref):
    o_ref[...] = jnp.dot(x_ref[...], w_ref[...],
                         preferred_element_type=jnp.float32)


def _fuse1_body(p_ref, y_ref, b1a_ref, w1b_ref, b1b_ref, w2a_ref, o_ref):
    t = jnp.maximum(p_ref[0] + p_ref[1] + y_ref[...] + b1a_ref[...], 0.0)
    h = jnp.maximum(
        jnp.dot(t, w1b_ref[...], preferred_element_type=jnp.float32)
        + b1b_ref[...], 0.0)
    o_ref[...] = jnp.dot(h, w2a_ref[...], preferred_element_type=jnp.float32)


def _fuse2_body(p_ref, y_ref, b2a_ref, w2b_ref, b2b_ref, batch_ref,
                wfc_ref, bfc_ref, o_ref):
    t = jnp.maximum(p_ref[0] + p_ref[1] + y_ref[...] + b2a_ref[...], 0.0)
    h = jnp.maximum(
        jnp.dot(t, w2b_ref[...], preferred_element_type=jnp.float32)
        + b2b_ref[...], 0.0)
    onehot = (batch_ref[...] ==
              lax.broadcasted_iota(jnp.int32, (G, N), 0)).astype(jnp.float32)
    sums = jnp.dot(onehot, h, preferred_element_type=jnp.float32)
    counts = jnp.dot(onehot, jnp.ones((N, 1), jnp.float32),
                     preferred_element_type=jnp.float32)
    g = sums / jnp.maximum(counts, 1.0)
    o_ref[...] = (jnp.dot(g, wfc_ref[...], preferred_element_type=jnp.float32)
                  + bfc_ref[...])


def kernel(x, edge_index, batch, W1a, b1a, W1b, b1b, W2a, b2a, W2b, b2b,
           Wfc, bfc):
    src = edge_index[0]
    dst = edge_index[1]
    pad = E_PAD - E
    src_p = jnp.concatenate([src, jnp.zeros((pad,), jnp.int32)]
                            ).reshape(NW, CH, K)
    dst_p = jnp.concatenate([dst, jnp.full((pad,), N, jnp.int32)]
                            ).reshape(NW, CH, K)
    zeros = jnp.zeros((N_PAD, H), jnp.float32)
    batch2d = batch.reshape(1, N)

    y1 = pl.pallas_call(
        _proj_body,
        out_shape=jax.ShapeDtypeStruct((N, H), jnp.float32),
    )(x, W1a)

    p1 = _segsum(y1, src_p, dst_p, zeros)

    y2 = pl.pallas_call(
        _fuse1_body,
        out_shape=jax.ShapeDtypeStruct((N, H), jnp.float32),
    )(p1, y1, b1a.reshape(1, H), W1b, b1b.reshape(1, H), W2a)

    p2 = _segsum(y2, src_p, dst_p, zeros)

    out = pl.pallas_call(
        _fuse2_body,
        out_shape=jax.ShapeDtypeStruct((G, C), jnp.float32),
    )(p2, y2, b2a.reshape(1, H), W2b, b2b.reshape(1, H), batch2d,
      Wfc, bfc.reshape(1, C))

    return out


# speedup vs baseline: 1.1101x; 1.1101x over previous
import functools

import jax
import jax.numpy as jnp
from jax import lax
from jax.experimental import pallas as pl
from jax.experimental.pallas import tpu as pltpu
from jax.experimental.pallas import tpu_sc as plsc

N, E, D, H, C, G = 10000, 320000, 128, 32, 2, 64

NC, NS = 2, 16
NW = NC * NS
K = 80
NB = 5
CH = E // (NW * K)
N_PAD = N


def _segsum_body(y_hbm, eidx_hbm, zeros_hbm, out_hbm,
                 src_v, dst_v, rows_v, acc, y_sh, gsems):
    c = lax.axis_index("c")
    s = lax.axis_index("s")
    wid = s * NC + c

    @pl.when(s == 0)
    def _zero():
        pltpu.sync_copy(zeros_hbm, acc)

    @pl.when(s == 1)
    def _stage():
        pltpu.sync_copy(y_hbm, y_sh)

    plsc.subcore_barrier()

    pltpu.sync_copy(eidx_hbm.at[0, wid], src_v)
    pltpu.sync_copy(eidx_hbm.at[1, wid], dst_v)

    for b in range(NB):
        pltpu.async_copy(y_sh.at[src_v.at[b]], rows_v.at[b], gsems[b])

    def body(g, carry):
        for b in range(NB):
            j = g * NB + b
            pltpu.make_async_copy(y_sh.at[src_v.at[j]], rows_v.at[b],
                                  gsems[b]).wait()
            pltpu.sync_copy(rows_v.at[b], acc.at[dst_v.at[j]], add=True)

            @pl.when(j + NB < CH)
            def _refill():
                pltpu.async_copy(y_sh.at[src_v.at[j + NB]], rows_v.at[b],
                                 gsems[b])
        return carry

    lax.fori_loop(0, CH // NB, body, 0)

    plsc.subcore_barrier()

    @pl.when(s == 0)
    def _writeout():
        pltpu.sync_copy(acc.at[pl.ds(0, N)], out_hbm.at[c])


_segsum = functools.partial(
    pl.kernel,
    out_type=jax.ShapeDtypeStruct((2, N, H), jnp.float32),
    mesh=plsc.VectorSubcoreMesh(core_axis_name="c", subcore_axis_name="s",
                                num_cores=NC, num_subcores=NS),
    compiler_params=pltpu.CompilerParams(use_tc_tiling_on_sc=False),
    scratch_types=[
        pltpu.VMEM((CH, K), jnp.int32),
        pltpu.VMEM((CH, K), jnp.int32),
        pltpu.VMEM((NB, K, H), jnp.float32),
        pltpu.VMEM_SHARED((N_PAD, H), jnp.float32),
        pltpu.VMEM_SHARED((N, H), jnp.float32),
        [pltpu.SemaphoreType.DMA] * NB,
    ],
)(_segsum_body)


def _proj_body(x_ref, w_ref, o_ref):
    o_ref[...] = jnp.dot(x_ref[...], w_ref[...],
                         preferred_element_type=jnp.float32)


def _fuse1_body(p_ref, y_ref, b1a_ref, w1b_ref, b1b_ref, w2a_ref, o_ref):
    t = jnp.maximum(p_ref[0] + p_ref[1] + y_ref[...] + b1a_ref[...], 0.0)
    h = jnp.maximum(
        jnp.dot(t, w1b_ref[...], preferred_element_type=jnp.float32)
        + b1b_ref[...], 0.0)
    o_ref[...] = jnp.dot(h, w2a_ref[...], preferred_element_type=jnp.float32)


def _fuse2_body(p_ref, y_ref, b2a_ref, w2b_ref, b2b_ref, batch_ref,
                wfc_ref, bfc_ref, o_ref):
    t = jnp.maximum(p_ref[0] + p_ref[1] + y_ref[...] + b2a_ref[...], 0.0)
    h = jnp.maximum(
        jnp.dot(t, w2b_ref[...], preferred_element_type=jnp.float32)
        + b2b_ref[...], 0.0)
    onehot = (batch_ref[...] ==
              lax.broadcasted_iota(jnp.int32, (G, N), 0)).astype(jnp.float32)
    sums = jnp.dot(onehot, h, preferred_element_type=jnp.float32)
    counts = jnp.dot(onehot, jnp.ones((N, 1), jnp.float32),
                     preferred_element_type=jnp.float32)
    g = sums / jnp.maximum(counts, 1.0)
    o_ref[...] = (jnp.dot(g, wfc_ref[...], preferred_element_type=jnp.float32)
                  + bfc_ref[...])


def kernel(x, edge_index, batch, W1a, b1a, W1b, b1b, W2a, b2a, W2b, b2b,
           Wfc, bfc):
    eidx = edge_index.reshape(2, NW, CH, K)
    zeros = jnp.zeros((N_PAD, H), jnp.float32)
    batch2d = batch.reshape(1, N)

    y1 = pl.pallas_call(
        _proj_body,
        out_shape=jax.ShapeDtypeStruct((N, H), jnp.float32),
    )(x, W1a)

    p1 = _segsum(y1, eidx, zeros)

    y2 = pl.pallas_call(
        _fuse1_body,
        out_shape=jax.ShapeDtypeStruct((N, H), jnp.float32),
    )(p1, y1, b1a.reshape(1, H), W1b, b1b.reshape(1, H), W2a)

    p2 = _segsum(y2, eidx, zeros)

    out = pl.pallas_call(
        _fuse2_body,
        out_shape=jax.ShapeDtypeStruct((G, C), jnp.float32),
    )(p2, y2, b2a.reshape(1, H), W2b, b2b.reshape(1, H), batch2d,
      Wfc, bfc.reshape(1, C))

    return out


# speedup vs baseline: 1.3356x; 1.2031x over previous
import functools

import jax
import jax.numpy as jnp
from jax import lax
from jax.experimental import pallas as pl
from jax.experimental.pallas import tpu as pltpu
from jax.experimental.pallas import tpu_sc as plsc

N, E, D, H, C, G = 10000, 320000, 128, 32, 2, 64

R = 2504
M = 4 * R
Q3 = N - 3 * R

NC, NS = 2, 16
NW = NC * NS
K = 80
NB = 5
CH = E // (NW * K)


def _segsum_body(y_hbm, eidx_hbm, zeros_hbm, out_hbm,
                 src_v, dst_v, rows_v, acc, y_sh, gsems):
    c = lax.axis_index("c")
    s = lax.axis_index("s")
    wid = s * NC + c

    @pl.when(s == 0)
    def _zero():
        pltpu.sync_copy(zeros_hbm, acc)

    @pl.when(s == 1)
    def _stage():
        pltpu.sync_copy(y_hbm, y_sh)

    plsc.subcore_barrier()

    pltpu.sync_copy(eidx_hbm.at[0, wid], src_v)
    pltpu.sync_copy(eidx_hbm.at[1, wid], dst_v)

    for b in range(NB):
        pltpu.async_copy(y_sh.at[src_v.at[b]], rows_v.at[b], gsems[b])

    def body(g, carry):
        for b in range(NB):
            j = g * NB + b
            pltpu.make_async_copy(y_sh.at[src_v.at[j]], rows_v.at[b],
                                  gsems[b]).wait()
            pltpu.sync_copy(rows_v.at[b], acc.at[dst_v.at[j]], add=True)

            @pl.when(j + NB < CH)
            def _refill():
                pltpu.async_copy(y_sh.at[src_v.at[j + NB]], rows_v.at[b],
                                 gsems[b])
        return carry

    lax.fori_loop(0, CH // NB, body, 0)

    plsc.subcore_barrier()

    @pl.when(s == 0)
    def _writeout():
        pltpu.sync_copy(acc, out_hbm.at[c])


_segsum = functools.partial(
    pl.kernel,
    out_type=jax.ShapeDtypeStruct((2, M, H), jnp.float32),
    mesh=plsc.VectorSubcoreMesh(core_axis_name="c", subcore_axis_name="s",
                                num_cores=NC, num_subcores=NS),
    compiler_params=pltpu.CompilerParams(use_tc_tiling_on_sc=False),
    scratch_types=[
        pltpu.VMEM((CH, K), jnp.int32),
        pltpu.VMEM((CH, K), jnp.int32),
        pltpu.VMEM((NB, K, H), jnp.float32),
        pltpu.VMEM_SHARED((M, H), jnp.float32),
        pltpu.VMEM_SHARED((M, H), jnp.float32),
        [pltpu.SemaphoreType.DMA] * NB,
    ],
)(_segsum_body)


def _proj_body(x_ref, w_ref, o_ref):
    for q in range(3):
        xq = x_ref[pl.ds(q * R, R), :]
        o_ref[:, pl.ds(32 * q, 32)] = jnp.dot(
            xq, w_ref[...], preferred_element_type=jnp.float32)
    x3 = x_ref[pl.ds(3 * R, Q3), :]
    y3 = jnp.dot(x3, w_ref[...], preferred_element_type=jnp.float32)
    o_ref[pl.ds(0, Q3), pl.ds(96, 32)] = y3
    o_ref[pl.ds(Q3, R - Q3), pl.ds(96, 32)] = jnp.zeros(
        (R - Q3, 32), jnp.float32)


def _fuse1_body(p_ref, y_ref, b1a_ref, w1b_ref, b1b_ref, w2a_ref, o_ref):
    t = jnp.maximum(p_ref[0] + p_ref[1] + y_ref[...] + b1a_ref[...], 0.0)
    h = jnp.maximum(
        jnp.dot(t, w1b_ref[...], preferred_element_type=jnp.float32)
        + b1b_ref[...], 0.0)
    o_ref[...] = jnp.dot(h, w2a_ref[...], preferred_element_type=jnp.float32)


def _fuse2_body(p_ref, y_ref, b2a_ref, w2b_ref, b2b_ref, batchp_ref,
                wfc_ref, bfc_ref, o_ref):
    t = jnp.maximum(p_ref[0] + p_ref[1] + y_ref[...] + b2a_ref[...], 0.0)
    h = jnp.maximum(
        jnp.dot(t, w2b_ref[...], preferred_element_type=jnp.float32)
        + b2b_ref[...], 0.0)
    sums = jnp.zeros((G, H), jnp.float32)
    counts = jnp.zeros((G, 1), jnp.float32)
    ones = jnp.ones((R, 1), jnp.float32)
    for q in range(4):
        bq = batchp_ref[pl.ds(q, 1), :]
        pq = (bq == lax.broadcasted_iota(jnp.int32, (G, R), 0)
              ).astype(jnp.float32)
        hq = h[:, 32 * q:32 * (q + 1)]
        sums = sums + jnp.dot(pq, hq, preferred_element_type=jnp.float32)
        counts = counts + jnp.dot(pq, ones,
                                  preferred_element_type=jnp.float32)
    g = sums / jnp.maximum(counts, 1.0)
    o_ref[...] = (jnp.dot(g, wfc_ref[...], preferred_element_type=jnp.float32)
                  + bfc_ref[...])


def _block_diag4(w):
    d0, d1 = w.shape
    z = jnp.zeros((d0, d1), w.dtype)
    rows = [jnp.concatenate([w if i == q else z for i in range(4)], axis=1)
            for q in range(4)]
    return jnp.concatenate(rows, axis=0)


def kernel(x, edge_index, batch, W1a, b1a, W1b, b1b, W2a, b2a, W2b, b2b,
           Wfc, bfc):
    q = edge_index // R
    slots = 4 * edge_index - (4 * R - 1) * q
    eidx = slots.reshape(2, NW, CH, K)

    zeros = jnp.zeros((M, H), jnp.float32)
    batchp = jnp.pad(batch, (0, M - N), constant_values=-1).reshape(4, R)

    w1b_bd = _block_diag4(W1b)
    w2a_bd = _block_diag4(W2a)
    w2b_bd = _block_diag4(W2b)
    b1a_t = jnp.tile(b1a, 4).reshape(1, 4 * H)
    b1b_t = jnp.tile(b1b, 4).reshape(1, 4 * H)
    b2a_t = jnp.tile(b2a, 4).reshape(1, 4 * H)
    b2b_t = jnp.tile(b2b, 4).reshape(1, 4 * H)

    y1p = pl.pallas_call(
        _proj_body,
        out_shape=jax.ShapeDtypeStruct((R, 4 * H), jnp.float32),
    )(x, W1a)

    p1 = _segsum(y1p.reshape(M, H), eidx, zeros)

    y2p = pl.pallas_call(
        _fuse1_body,
        out_shape=jax.ShapeDtypeStruct((R, 4 * H), jnp.float32),
    )(p1.reshape(2, R, 4 * H), y1p, b1a_t, w1b_bd, b1b_t, w2a_bd)

    p2 = _segsum(y2p.reshape(M, H), eidx, zeros)

    out = pl.pallas_call(
        _fuse2_body,
        out_shape=jax.ShapeDtypeStruct((G, C), jnp.float32),
    )(p2.reshape(2, R, 4 * H), y2p, b2a_t, w2b_bd, b2b_t, batchp,
      Wfc, bfc.reshape(1, C))

    return out


# speedup vs baseline: 1.3987x; 1.0472x over previous
import functools

import jax
import jax.numpy as jnp
from jax import lax
from jax.experimental import pallas as pl
from jax.experimental.pallas import tpu as pltpu
from jax.experimental.pallas import tpu_sc as plsc

N, E, D, H, C, G = 10000, 320000, 128, 32, 2, 64

R = 2504
M = 4 * R
Q3 = N - 3 * R

NC, NS = 2, 16
NW = NC * NS
K = 80
NB = 5
CH = E // (NW * K)


def _segsum_body(y_hbm, eidx_hbm, zeros_hbm, out_hbm,
                 src_v, dst_v, rows_v, acc, y_sh, gsems, ssems):
    c = lax.axis_index("c")
    s = lax.axis_index("s")
    wid = s * NC + c

    @pl.when(s == 0)
    def _zero():
        pltpu.sync_copy(zeros_hbm, acc)

    @pl.when(s == 1)
    def _stage():
        pltpu.sync_copy(y_hbm, y_sh)

    plsc.subcore_barrier()

    pltpu.sync_copy(eidx_hbm.at[0, wid], src_v)
    pltpu.sync_copy(eidx_hbm.at[1, wid], dst_v)

    for b in range(NB):
        pltpu.async_copy(y_sh.at[src_v.at[b]], rows_v.at[b], gsems[b])

    def body(g, carry):
        for b in range(NB):
            j = g * NB + b
            bp = (b - 1) % NB
            jp = j - 1

            @pl.when(jnp.logical_and(jp >= 0, jp + NB < CH))
            def _refill():
                pltpu.make_async_copy(rows_v.at[bp], acc.at[dst_v.at[jp]],
                                      ssems[bp]).wait()
                pltpu.async_copy(y_sh.at[src_v.at[jp + NB]], rows_v.at[bp],
                                 gsems[bp])

            pltpu.make_async_copy(y_sh.at[src_v.at[j]], rows_v.at[b],
                                  gsems[b]).wait()
            pltpu.async_copy(rows_v.at[b], acc.at[dst_v.at[j]], ssems[b],
                             add=True)
        return carry

    lax.fori_loop(0, CH // NB, body, 0)
    for b in range(NB):
        j = CH - NB + b
        pltpu.make_async_copy(rows_v.at[b], acc.at[dst_v.at[j]],
                              ssems[b]).wait()

    plsc.subcore_barrier()

    @pl.when(s == 0)
    def _writeout():
        pltpu.sync_copy(acc, out_hbm.at[c])


_segsum = functools.partial(
    pl.kernel,
    out_type=jax.ShapeDtypeStruct((2, M, H), jnp.float32),
    mesh=plsc.VectorSubcoreMesh(core_axis_name="c", subcore_axis_name="s",
                                num_cores=NC, num_subcores=NS),
    compiler_params=pltpu.CompilerParams(use_tc_tiling_on_sc=False),
    scratch_types=[
        pltpu.VMEM((CH, K), jnp.int32),
        pltpu.VMEM((CH, K), jnp.int32),
        pltpu.VMEM((NB, K, H), jnp.float32),
        pltpu.VMEM_SHARED((M, H), jnp.float32),
        pltpu.VMEM_SHARED((M, H), jnp.float32),
        [pltpu.SemaphoreType.DMA] * NB,
        [pltpu.SemaphoreType.DMA] * NB,
    ],
)(_segsum_body)


def _proj_body(x_ref, w_ref, o_ref):
    for q in range(3):
        xq = x_ref[pl.ds(q * R, R), :]
        o_ref[:, pl.ds(32 * q, 32)] = jnp.dot(
            xq, w_ref[...], preferred_element_type=jnp.float32)
    x3 = x_ref[pl.ds(3 * R, Q3), :]
    y3 = jnp.dot(x3, w_ref[...], preferred_element_type=jnp.float32)
    o_ref[pl.ds(0, Q3), pl.ds(96, 32)] = y3
    o_ref[pl.ds(Q3, R - Q3), pl.ds(96, 32)] = jnp.zeros(
        (R - Q3, 32), jnp.float32)


def _fuse1_body(p_ref, y_ref, b1a_ref, w1b_ref, b1b_ref, w2a_ref, o_ref):
    t = jnp.maximum(p_ref[0] + p_ref[1] + y_ref[...] + b1a_ref[...], 0.0)
    h = jnp.maximum(
        jnp.dot(t, w1b_ref[...], preferred_element_type=jnp.float32)
        + b1b_ref[...], 0.0)
    o_ref[...] = jnp.dot(h, w2a_ref[...], preferred_element_type=jnp.float32)


def _fuse2_body(p_ref, y_ref, b2a_ref, w2b_ref, b2b_ref, batchp_ref,
                wfc_ref, bfc_ref, o_ref):
    t = jnp.maximum(p_ref[0] + p_ref[1] + y_ref[...] + b2a_ref[...], 0.0)
    h = jnp.maximum(
        jnp.dot(t, w2b_ref[...], preferred_element_type=jnp.float32)
        + b2b_ref[...], 0.0)
    sums = jnp.zeros((G, H), jnp.float32)
    counts = jnp.zeros((G, 1), jnp.float32)
    ones = jnp.ones((R, 1), jnp.float32)
    for q in range(4):
        bq = batchp_ref[pl.ds(q, 1), :]
        pq = (bq == lax.broadcasted_iota(jnp.int32, (G, R), 0)
              ).astype(jnp.float32)
        hq = h[:, 32 * q:32 * (q + 1)]
        sums = sums + jnp.dot(pq, hq, preferred_element_type=jnp.float32)
        counts = counts + jnp.dot(pq, ones,
                                  preferred_element_type=jnp.float32)
    g = sums / jnp.maximum(counts, 1.0)
    o_ref[...] = (jnp.dot(g, wfc_ref[...], preferred_element_type=jnp.float32)
                  + bfc_ref[...])


def _block_diag4(w):
    d0, d1 = w.shape
    z = jnp.zeros((d0, d1), w.dtype)
    rows = [jnp.concatenate([w if i == q else z for i in range(4)], axis=1)
            for q in range(4)]
    return jnp.concatenate(rows, axis=0)


def kernel(x, edge_index, batch, W1a, b1a, W1b, b1b, W2a, b2a, W2b, b2b,
           Wfc, bfc):
    e4 = edge_index.reshape(2, NW, CH, K)
    q = ((e4 >= R).astype(jnp.int32) + (e4 >= 2 * R).astype(jnp.int32)
         + (e4 >= 3 * R).astype(jnp.int32))
    eidx = 4 * e4 - (4 * R - 1) * q

    zeros = jnp.zeros((M, H), jnp.float32)
    batchp = jnp.pad(batch, (0, M - N), constant_values=-1).reshape(4, R)

    w1b_bd = _block_diag4(W1b)
    w2a_bd = _block_diag4(W2a)
    w2b_bd = _block_diag4(W2b)
    b1a_t = jnp.tile(b1a, 4).reshape(1, 4 * H)
    b1b_t = jnp.tile(b1b, 4).reshape(1, 4 * H)
    b2a_t = jnp.tile(b2a, 4).reshape(1, 4 * H)
    b2b_t = jnp.tile(b2b, 4).reshape(1, 4 * H)

    y1p = pl.pallas_call(
        _proj_body,
        out_shape=jax.ShapeDtypeStruct((R, 4 * H), jnp.float32),
    )(x, W1a)

    p1 = _segsum(y1p.reshape(M, H), eidx, zeros)

    y2p = pl.pallas_call(
        _fuse1_body,
        out_shape=jax.ShapeDtypeStruct((R, 4 * H), jnp.float32),
    )(p1.reshape(2, R, 4 * H), y1p, b1a_t, w1b_bd, b1b_t, w2a_bd)

    p2 = _segsum(y2p.reshape(M, H), eidx, zeros)

    out = pl.pallas_call(
        _fuse2_body,
        out_shape=jax.ShapeDtypeStruct((G, C), jnp.float32),
    )(p2.reshape(2, R, 4 * H), y2p, b2a_t, w2b_bd, b2b_t, batchp,
      Wfc, bfc.reshape(1, C))

    return out
